# baseline (speedup 1.0000x reference)
import functools

import jax
import jax.numpy as jnp
from jax import lax
from jax.experimental import pallas as pl
from jax.experimental.pallas import tpu as pltpu
from jax.experimental.pallas import tpu_sc as plsc

_B = 8
_MAX_LEN = 2048
_NW = 32
_CHUNK = _B * _MAX_LEN // _NW
_VPW = _CHUNK // 16


def _sc_body(pieces_hbm, len_hbm, out_hbm, rs_hbm, len_v, buf_v, rs_v):
    wid = lax.axis_index("s") * 2 + lax.axis_index("c")
    base = wid * _CHUNK
    row = base // _MAX_LEN

    pltpu.sync_copy(len_hbm, len_v)
    pltpu.sync_copy(pieces_hbm.at[pl.ds(base, _CHUNK)], buf_v)

    lane = lax.broadcasted_iota(jnp.int32, (16,), 0)
    lv = len_v[...]
    row_len = jnp.sum(jnp.where(lane == row, lv, 0))
    col0 = base % _MAX_LEN
    for j in range(_VPW):
        col = lane + (col0 + j * 16)
        vals = buf_v[pl.ds(j * 16, 16)]
        buf_v[pl.ds(j * 16, 16)] = jnp.where(col < row_len, vals, 0)

    pltpu.sync_copy(buf_v, out_hbm.at[pl.ds(base, _CHUNK)])

    @pl.when(wid == 0)
    def _():
        excl = plsc.cumsum(lv) - lv
        rs_v[...] = jnp.where(lane < _B, excl, jnp.sum(lv))
        pltpu.sync_copy(rs_v, rs_hbm)


@functools.partial(jax.jit, static_argnames=())
def kernel(pieces, lengths):
    mesh = plsc.VectorSubcoreMesh(core_axis_name="c", subcore_axis_name="s")
    flat = pieces.reshape(_B * _MAX_LEN)
    len_pad = jnp.zeros((16,), jnp.int32).at[:_B].set(lengths)
    out_flat, rs = pl.kernel(
        _sc_body,
        out_type=[
            jax.ShapeDtypeStruct((_B * _MAX_LEN,), jnp.int32),
            jax.ShapeDtypeStruct((16,), jnp.int32),
        ],
        mesh=mesh,
        scratch_types=[
            pltpu.VMEM((16,), jnp.int32),
            pltpu.VMEM((_CHUNK,), jnp.int32),
            pltpu.VMEM((16,), jnp.int32),
        ],
        compiler_params=pltpu.CompilerParams(needs_layout_passes=False),
    )(flat, len_pad)
    return out_flat.reshape(_B, _MAX_LEN), rs[: _B + 1]


# speedup vs baseline: 1.1091x; 1.1091x over previous
import functools

import jax
import jax.numpy as jnp
from jax import lax
from jax.experimental import pallas as pl
from jax.experimental.pallas import tpu as pltpu
from jax.experimental.pallas import tpu_sc as plsc

_B = 8
_MAX_LEN = 2048
_NW = 32
_CHUNK = _B * _MAX_LEN // _NW
_VPW = _CHUNK // 16


_NW1 = 16
_CHUNK1 = _B * _MAX_LEN // _NW1


def _sc_body(pieces_hbm, len_hbm, out_hbm, rs_hbm, len_v, buf_v, rs_v):
    wid = lax.axis_index("s")
    base = wid * _CHUNK1
    row = base // _MAX_LEN

    pltpu.sync_copy(len_hbm, len_v.at[pl.ds(0, _B)])
    pltpu.sync_copy(pieces_hbm.at[pl.ds(base, _CHUNK1)], buf_v)

    lane = lax.broadcasted_iota(jnp.int32, (16,), 0)
    lv = jnp.where(lane < _B, len_v[...], 0)
    row_len = jnp.sum(jnp.where(lane == row, lv, 0))
    col0 = base % _MAX_LEN
    for j in range(_CHUNK1 // 16):
        col = lane + (col0 + j * 16)
        vals = buf_v[pl.ds(j * 16, 16)]
        buf_v[pl.ds(j * 16, 16)] = jnp.where(col < row_len, vals, 0)

    pltpu.sync_copy(buf_v, out_hbm.at[pl.ds(base, _CHUNK1)])

    @pl.when(wid == 0)
    def _():
        excl = plsc.cumsum(lv) - lv
        rs_v[...] = jnp.where(lane < _B, excl, jnp.sum(lv))
        pltpu.sync_copy(rs_v.at[pl.ds(0, _B + 1)], rs_hbm)


@functools.partial(jax.jit, static_argnames=())
def kernel(pieces, lengths):
    mesh = plsc.VectorSubcoreMesh(
        core_axis_name="c", subcore_axis_name="s", num_cores=1
    )
    flat = pieces.reshape(_B * _MAX_LEN)
    out_flat, rs = pl.kernel(
        _sc_body,
        out_type=[
            jax.ShapeDtypeStruct((_B * _MAX_LEN,), jnp.int32),
            jax.ShapeDtypeStruct((_B + 1,), jnp.int32),
        ],
        mesh=mesh,
        scratch_types=[
            pltpu.VMEM((16,), jnp.int32),
            pltpu.VMEM((_CHUNK1,), jnp.int32),
            pltpu.VMEM((16,), jnp.int32),
        ],
        compiler_params=pltpu.CompilerParams(needs_layout_passes=False),
    )(flat, lengths)
    return out_flat.reshape(_B, _MAX_LEN), rs
